# baseline (speedup 1.0000x reference)
import jax
import jax.numpy as jnp
from jax import lax
from jax.experimental import pallas as pl
from jax.experimental.pallas import tpu as pltpu

B, S, V, D, C, E, H = 64, 2048, 6, 128, 40, 64, 512
RPS = 2
NSTEP = B // RPS


def _dotT(a, b):
    return lax.dot_general(a, b, (((1,), (1,)), ((), ())),
                           preferred_element_type=jnp.float32)


def _sel6(masks, operands, init):
    acc = init
    for m, o in zip(masks, operands):
        acc = jnp.where(m, o, acc)
    return acc


def _fused_kernel(tok_pair_ref, tab_ref, w_ref,
                  xd_ref, xw_ref, xb_ref, pw_ref, b2_ref, g_ref, bt_ref,
                  pos_ref,
                  out_ref,
                  tp_ref, eh_ref, ehg_ref, pp_ref, ppg_ref, g1_ref,
                  mq_pp_ref, g2t_ref, meh_ref, qeh_ref, lhs_ref, rhs_ref):
    step = pl.program_id(0)

    @pl.when(step == 0)
    def _init():
        g = g_ref[...]
        tp = _dotT(tab_ref[...], w_ref[...])
        tp_ref[...] = tp
        e = _dotT(xd_ref[...], xw_ref[...]) + xb_ref[...]
        eh = _dotT(e, pw_ref[...]) + b2_ref[...]
        eh_ref[...] = eh
        ehg_ref[...] = eh * g
        pp = _dotT(pos_ref[...], w_ref[...])
        pp_ref[...] = pp
        ppg_ref[...] = pp * g
        g1_ref[...] = _dotT(tp, pp)
        g2t_ref[...] = _dotT(eh, tp)
        ones_h = jnp.ones((1, H), jnp.float32)
        mq_pp_ref[0:1, :] = _dotT(ones_h, pp) * (1.0 / H)
        mq_pp_ref[1:2, :] = _dotT(ones_h, pp * pp)
        meh_ref[...] = jnp.mean(eh, axis=1, keepdims=True)
        qeh_ref[...] = jnp.sum(eh * eh, axis=1, keepdims=True)
        tpg = tp * g
        for k in range(RPS):
            rhs_ref[k, 0:8, :] = tpg
            rhs_ref[k, 8:9, :] = -g
            rhs_ref[k, 9:10, :] = bt_ref[...]
            rhs_ref[k, 10:16, :] = jnp.zeros((6, H), jnp.float32)
            lhs_ref[k, 8:16, :] = jnp.zeros((8, S), jnp.float32)
            lhs_ref[k, 9:10, :] = jnp.ones((1, S), jnp.float32)

    tok_pair = tok_pair_ref[0, 0]
    b0 = step * RPS
    def _pair(ref):
        return jnp.concatenate(
            [ref[pl.ds(b0 + k, 1), :] for k in range(RPS)], axis=0)
    eh_pair = _pair(eh_ref)
    g3_pair = _dotT(eh_pair, pp_ref[...])
    g2_pair = _pair(g2t_ref)
    meh_pair = _pair(meh_ref)
    qeh_pair = _pair(qeh_ref)
    tp = tp_ref[...]
    mtp = jnp.mean(tp, axis=1, keepdims=True)
    qtp = jnp.sum(tp * tp, axis=1, keepdims=True)
    g1 = g1_ref[...]
    masks = [tok_pair == v for v in range(1, V)]
    mu = (_sel6(masks, [mtp[v:v + 1, 0:1] for v in range(1, V)],
                mtp[0:1, 0:1])
          + meh_pair + mq_pp_ref[0:1, :])
    ss = (_sel6(masks, [qtp[v:v + 1, 0:1] for v in range(1, V)],
                qtp[0:1, 0:1])
          + 2.0 * _sel6(masks, [g1[v:v + 1, :] for v in range(1, V)],
                        g1[0:1, :])
          + 2.0 * _sel6(masks, [g2_pair[:, v:v + 1] for v in range(1, V)],
                        g2_pair[:, 0:1])
          + qeh_pair + mq_pp_ref[1:2, :] + 2.0 * g3_pair)
    var = ss * (1.0 / H) - mu * mu
    r_pair = lax.rsqrt(var + 1e-5)
    mr_pair = mu * r_pair
    rt_pair = jnp.concatenate(
        [r_pair, jnp.zeros((8 - RPS, S), jnp.float32)], axis=0).T

    iota = lax.broadcasted_iota(jnp.int32, (8, S), 0)
    ppg = ppg_ref[...]
    for k in range(RPS):
        tok = tok_pair[k:k + 1, :]
        r_row = r_pair[k:k + 1, :]
        lhs_ref[k, 0:8, :] = jnp.where(iota == tok, r_row, 0.0)
        lhs_ref[k, 8:9, :] = mr_pair[k:k + 1, :]
        lhs_ref[k, 10:11, :] = r_row
        rhs_ref[k, 10:11, :] = ehg_ref[pl.ds(b0 + k, 1), :]
        y = lax.dot_general(lhs_ref[k], rhs_ref[k],
                            (((0,), (0,)), ((), ())),
                            preferred_element_type=jnp.float32)
        r_col = rt_pair[:, k:k + 1]
        out_ref[k] = y + ppg * r_col


def kernel(dna_tokens, expr_data, dna_table, pos_enc, expr_W, expr_b,
           dna_proj_W, dna_proj_b, expr_proj_W, expr_proj_b, ln_gamma, ln_beta):
    toks = dna_tokens.astype(jnp.int32)
    tok_pair = toks.reshape(NSTEP, 1, RPS, S)
    tab8 = jnp.pad(dna_table, ((0, 8 - V), (0, 0)))
    xd = jnp.pad(expr_data, ((0, 0), (0, 128 - C)))
    xw = jnp.pad(expr_W, ((0, 0), (0, 128 - C)))
    xb = expr_b.reshape(1, E)
    b2 = (expr_proj_b + dna_proj_b).reshape(1, H)
    g2 = ln_gamma.reshape(1, H)
    bt2 = ln_beta.reshape(1, H)

    out = pl.pallas_call(
        _fused_kernel,
        grid=(NSTEP,),
        in_specs=[
            pl.BlockSpec((1, 1, RPS, S), lambda i: (i, 0, 0, 0)),
            pl.BlockSpec((8, D), lambda i: (0, 0)),
            pl.BlockSpec((H, D), lambda i: (0, 0)),
            pl.BlockSpec((B, 128), lambda i: (0, 0)),
            pl.BlockSpec((E, 128), lambda i: (0, 0)),
            pl.BlockSpec((1, E), lambda i: (0, 0)),
            pl.BlockSpec((H, E), lambda i: (0, 0)),
            pl.BlockSpec((1, H), lambda i: (0, 0)),
            pl.BlockSpec((1, H), lambda i: (0, 0)),
            pl.BlockSpec((1, H), lambda i: (0, 0)),
            pl.BlockSpec((S, D), lambda i: (0, 0)),
        ],
        out_specs=pl.BlockSpec((RPS, S, H), lambda i: (i, 0, 0)),
        out_shape=jax.ShapeDtypeStruct((B, S, H), jnp.float32),
        scratch_shapes=[
            pltpu.VMEM((8, H), jnp.float32),
            pltpu.VMEM((B, H), jnp.float32),
            pltpu.VMEM((B, H), jnp.float32),
            pltpu.VMEM((S, H), jnp.float32),
            pltpu.VMEM((S, H), jnp.float32),
            pltpu.VMEM((8, S), jnp.float32),
            pltpu.VMEM((2, S), jnp.float32),
            pltpu.VMEM((B, 8), jnp.float32),
            pltpu.VMEM((B, 1), jnp.float32),
            pltpu.VMEM((B, 1), jnp.float32),
            pltpu.VMEM((RPS, 16, S), jnp.float32),
            pltpu.VMEM((RPS, 16, H), jnp.float32),
        ],
    )(tok_pair, tab8, dna_proj_W, xd, xw, xb, expr_proj_W,
      b2, g2, bt2, pos_enc)
    return out


# speedup vs baseline: 1.0535x; 1.0535x over previous
import jax
import jax.numpy as jnp
from jax import lax
from jax.experimental import pallas as pl
from jax.experimental.pallas import tpu as pltpu

B, S, V, D, C, E, H = 64, 2048, 6, 128, 40, 64, 512
RPS = 2
NSTEP = B // RPS


def _dotT(a, b, fast=False):
    return lax.dot_general(a, b, (((1,), (1,)), ((), ())),
                           precision=(lax.Precision.DEFAULT if fast
                                      else lax.Precision.HIGHEST),
                           preferred_element_type=jnp.float32)


def _sel6(masks, operands, init):
    acc = init
    for m, o in zip(masks, operands):
        acc = jnp.where(m, o, acc)
    return acc


def _fused_kernel(tok_lane_ref, tok_all_ref, pos_ref, tab_ref, w_ref,
                  xd_ref, xw_ref, xb_ref, pw_ref, b2_ref, g_ref, bt_ref,
                  out_ref,
                  tp_ref, ehg_ref, ppg_ref, mr_ref, r_ref, rt_ref,
                  lhs_ref, rhs_ref):
    step = pl.program_id(0)

    @pl.when(step == 0)
    def _init():
        g = g_ref[...]
        tp = _dotT(tab_ref[...], w_ref[...])
        tp_ref[...] = tp
        e = _dotT(xd_ref[...], xw_ref[...]) + xb_ref[...]
        eh = _dotT(e, pw_ref[...]) + b2_ref[...]
        ehg_ref[...] = eh * g
        pp = _dotT(pos_ref[...], w_ref[...])
        ppg_ref[...] = pp * g
        g1 = _dotT(tp, pp, fast=True)
        g2t = _dotT(eh, tp)
        g3t = _dotT(eh, pp, fast=True)
        ones_h = jnp.ones((1, H), jnp.float32)
        m_pp = _dotT(ones_h, pp, fast=True) * (1.0 / H)
        q_pp = _dotT(ones_h, pp * pp, fast=True)
        meh = jnp.mean(eh, axis=1, keepdims=True)
        qeh = jnp.sum(eh * eh, axis=1, keepdims=True)
        tok_all = tok_all_ref[...]
        masks = [tok_all == v for v in range(1, V)]
        mtp = jnp.mean(tp, axis=1, keepdims=True)
        qtp = jnp.sum(tp * tp, axis=1, keepdims=True)
        mu = (_sel6(masks, [mtp[v:v + 1, 0:1] for v in range(1, V)],
                    mtp[0:1, 0:1])
              + meh + m_pp)
        ss = (_sel6(masks, [qtp[v:v + 1, 0:1] for v in range(1, V)],
                    qtp[0:1, 0:1])
              + 2.0 * _sel6(masks, [g1[v:v + 1, :] for v in range(1, V)],
                            g1[0:1, :])
              + 2.0 * _sel6(masks, [g2t[:, v:v + 1] for v in range(1, V)],
                            g2t[:, 0:1])
              + qeh + q_pp + 2.0 * g3t)
        var = ss * (1.0 / H) - mu * mu
        r = lax.rsqrt(var + 1e-5)
        r_ref[...] = r
        mr_ref[...] = mu * r
        rt_ref[...] = r.T
        tpg = tp * g
        for k in range(RPS):
            rhs_ref[k, 0:8, :] = tpg
            rhs_ref[k, 8:9, :] = -g
            rhs_ref[k, 9:10, :] = bt_ref[...]
            rhs_ref[k, 10:16, :] = jnp.zeros((6, H), jnp.float32)
            lhs_ref[k, 8:16, :] = jnp.zeros((8, S), jnp.float32)
            lhs_ref[k, 9:10, :] = jnp.ones((1, S), jnp.float32)

    iota = lax.broadcasted_iota(jnp.int32, (8, S), 0)
    iota_b = lax.broadcasted_iota(jnp.int32, (1, B), 1)
    ppg = ppg_ref[...]
    for k in range(RPS):
        bi = step * RPS + k
        tok = tok_lane_ref[k]
        r_row = r_ref[pl.ds(bi, 1), :]
        lhs_ref[k, 0:8, :] = jnp.where(iota == tok, r_row, 0.0)
        lhs_ref[k, 8:9, :] = mr_ref[pl.ds(bi, 1), :]
        lhs_ref[k, 10:11, :] = r_row
        rhs_ref[k, 10:11, :] = ehg_ref[pl.ds(bi, 1), :]
        y = lax.dot_general(lhs_ref[k], rhs_ref[k],
                            (((0,), (0,)), ((), ())),
                            preferred_element_type=jnp.float32)
        ohb = (iota_b == bi).astype(jnp.float32)
        r_col = jnp.sum(rt_ref[...] * ohb, axis=1, keepdims=True)
        out_ref[k] = y + ppg * r_col


def kernel(dna_tokens, expr_data, dna_table, pos_enc, expr_W, expr_b,
           dna_proj_W, dna_proj_b, expr_proj_W, expr_proj_b, ln_gamma, ln_beta):
    toks = dna_tokens.astype(jnp.int32)
    tok_lane = toks.reshape(B, 1, S)
    tab8 = jnp.pad(dna_table, ((0, 8 - V), (0, 0)))
    xd = jnp.pad(expr_data, ((0, 0), (0, 128 - C)))
    xw = jnp.pad(expr_W, ((0, 0), (0, 128 - C)))
    xb = expr_b.reshape(1, E)
    b2 = (expr_proj_b + dna_proj_b).reshape(1, H)
    g2 = ln_gamma.reshape(1, H)
    bt2 = ln_beta.reshape(1, H)

    out = pl.pallas_call(
        _fused_kernel,
        grid=(NSTEP,),
        in_specs=[
            pl.BlockSpec((RPS, 1, S), lambda i: (i, 0, 0)),
            pl.BlockSpec((B, S), lambda i: (0, 0)),
            pl.BlockSpec((S, D), lambda i: (0, 0)),
            pl.BlockSpec((8, D), lambda i: (0, 0)),
            pl.BlockSpec((H, D), lambda i: (0, 0)),
            pl.BlockSpec((B, 128), lambda i: (0, 0)),
            pl.BlockSpec((E, 128), lambda i: (0, 0)),
            pl.BlockSpec((1, E), lambda i: (0, 0)),
            pl.BlockSpec((H, E), lambda i: (0, 0)),
            pl.BlockSpec((1, H), lambda i: (0, 0)),
            pl.BlockSpec((1, H), lambda i: (0, 0)),
            pl.BlockSpec((1, H), lambda i: (0, 0)),
        ],
        out_specs=pl.BlockSpec((RPS, S, H), lambda i: (i, 0, 0)),
        out_shape=jax.ShapeDtypeStruct((B, S, H), jnp.float32),
        scratch_shapes=[
            pltpu.VMEM((8, H), jnp.float32),
            pltpu.VMEM((B, H), jnp.float32),
            pltpu.VMEM((S, H), jnp.float32),
            pltpu.VMEM((B, S), jnp.float32),
            pltpu.VMEM((B, S), jnp.float32),
            pltpu.VMEM((S, B), jnp.float32),
            pltpu.VMEM((RPS, 16, S), jnp.float32),
            pltpu.VMEM((RPS, 16, H), jnp.float32),
        ],
    )(tok_lane, toks, pos_enc, tab8, dna_proj_W, xd, xw, xb, expr_proj_W,
      b2, g2, bt2)
    return out


# speedup vs baseline: 1.0880x; 1.0327x over previous
import jax
import jax.numpy as jnp
from jax import lax
from jax.experimental import pallas as pl
from jax.experimental.pallas import tpu as pltpu

B, S, V, D, C, E, H = 64, 2048, 6, 128, 40, 64, 512
RPS = 2
NSTEP = B // RPS

_TAB0, _W0, _XD0, _XW0, _PW0, _POS0, _PACK_ROWS = 0, 8, 520, 584, 648, 1160, 3208


def _dotT(a, b):
    return lax.dot_general(a, b, (((1,), (1,)), ((), ())),
                           preferred_element_type=jnp.float32)


def _sel6(masks, operands, init):
    acc = init
    for m, o in zip(masks, operands):
        acc = jnp.where(m, o, acc)
    return acc


def _fused_kernel(tok_ref, pack_ref, vec_ref, out_ref,
                  tp_ref, ehg_ref, ppg_ref, mr_ref, r_ref, rt_ref,
                  lhs_ref, rhs_ref):
    step = pl.program_id(0)

    @pl.when(step == 0)
    def _init():
        g = vec_ref[1:2, :]
        bt = vec_ref[2:3, :]
        w = pack_ref[_W0:_W0 + H, :]
        tp = _dotT(pack_ref[_TAB0:_TAB0 + 8, :], w)
        tp_ref[...] = tp
        e = (_dotT(pack_ref[_XD0:_XD0 + B, :], pack_ref[_XW0:_XW0 + E, :])
             + vec_ref[3:4, 0:E])
        eh = _dotT(e, pack_ref[_PW0:_PW0 + H, 0:E]) + vec_ref[0:1, :]
        ehg_ref[...] = eh * g
        pp = _dotT(pack_ref[_POS0:_POS0 + S, :], w)
        ppg_ref[...] = pp * g
        g1 = _dotT(tp, pp)
        g2t = _dotT(eh, tp)
        g3t = _dotT(eh, pp)
        ones_h = jnp.ones((1, H), jnp.float32)
        m_pp = _dotT(ones_h, pp) * (1.0 / H)
        q_pp = _dotT(ones_h, pp * pp)
        meh = jnp.mean(eh, axis=1, keepdims=True)
        qeh = jnp.sum(eh * eh, axis=1, keepdims=True)
        tok_all = tok_ref[...]
        masks = [tok_all == v for v in range(1, V)]
        mtp = jnp.mean(tp, axis=1, keepdims=True)
        qtp = jnp.sum(tp * tp, axis=1, keepdims=True)
        mu = (_sel6(masks, [mtp[v:v + 1, 0:1] for v in range(1, V)],
                    mtp[0:1, 0:1])
              + meh + m_pp)
        ss = (_sel6(masks, [qtp[v:v + 1, 0:1] for v in range(1, V)],
                    qtp[0:1, 0:1])
              + 2.0 * _sel6(masks, [g1[v:v + 1, :] for v in range(1, V)],
                            g1[0:1, :])
              + 2.0 * _sel6(masks, [g2t[:, v:v + 1] for v in range(1, V)],
                            g2t[:, 0:1])
              + qeh + q_pp + 2.0 * g3t)
        var = ss * (1.0 / H) - mu * mu
        r = lax.rsqrt(var + 1e-5)
        r_ref[...] = r
        mr_ref[...] = mu * r
        rt_ref[...] = r.T
        tpg = tp * g
        for k in range(RPS):
            rhs_ref[k, 0:8, :] = tpg
            rhs_ref[k, 8:9, :] = -g
            rhs_ref[k, 9:10, :] = bt
            rhs_ref[k, 10:16, :] = jnp.zeros((6, H), jnp.float32)
            lhs_ref[k, 8:16, :] = jnp.zeros((8, S), jnp.float32)
            lhs_ref[k, 9:10, :] = jnp.ones((1, S), jnp.float32)

    iota = lax.broadcasted_iota(jnp.int32, (8, S), 0)
    iota_b = lax.broadcasted_iota(jnp.int32, (1, B), 1)
    ppg = ppg_ref[...]
    for k in range(RPS):
        bi = step * RPS + k
        tok = tok_ref[pl.ds(bi, 1), :]
        r_row = r_ref[pl.ds(bi, 1), :]
        lhs_ref[k, 0:8, :] = jnp.where(iota == tok, r_row, 0.0)
        lhs_ref[k, 8:9, :] = mr_ref[pl.ds(bi, 1), :]
        lhs_ref[k, 10:11, :] = r_row
        rhs_ref[k, 10:11, :] = ehg_ref[pl.ds(bi, 1), :]
        y = lax.dot_general(lhs_ref[k], rhs_ref[k],
                            (((0,), (0,)), ((), ())),
                            preferred_element_type=jnp.float32)
        ohb = (iota_b == bi).astype(jnp.float32)
        r_col = jnp.sum(rt_ref[...] * ohb, axis=1, keepdims=True)
        out_ref[k] = y + ppg * r_col


def kernel(dna_tokens, expr_data, dna_table, pos_enc, expr_W, expr_b,
           dna_proj_W, dna_proj_b, expr_proj_W, expr_proj_b, ln_gamma, ln_beta):
    toks = dna_tokens.astype(jnp.int32)
    pack = jnp.concatenate([
        jnp.pad(dna_table, ((0, 8 - V), (0, 0))),
        dna_proj_W,
        jnp.pad(expr_data, ((0, 0), (0, 128 - C))),
        jnp.pad(expr_W, ((0, 0), (0, 128 - C))),
        jnp.pad(expr_proj_W, ((0, 0), (0, 128 - E))),
        pos_enc,
    ], axis=0)
    vec = jnp.stack([
        expr_proj_b + dna_proj_b,
        ln_gamma,
        ln_beta,
        jnp.pad(expr_b, (0, H - E)),
    ], axis=0)

    out = pl.pallas_call(
        _fused_kernel,
        grid=(NSTEP,),
        in_specs=[
            pl.BlockSpec((B, S), lambda i: (0, 0)),
            pl.BlockSpec((_PACK_ROWS, 128), lambda i: (0, 0)),
            pl.BlockSpec((4, H), lambda i: (0, 0)),
        ],
        out_specs=pl.BlockSpec((RPS, S, H), lambda i: (i, 0, 0)),
        out_shape=jax.ShapeDtypeStruct((B, S, H), jnp.float32),
        scratch_shapes=[
            pltpu.VMEM((8, H), jnp.float32),
            pltpu.VMEM((B, H), jnp.float32),
            pltpu.VMEM((S, H), jnp.float32),
            pltpu.VMEM((B, S), jnp.float32),
            pltpu.VMEM((B, S), jnp.float32),
            pltpu.VMEM((S, B), jnp.float32),
            pltpu.VMEM((RPS, 16, S), jnp.float32),
            pltpu.VMEM((RPS, 16, H), jnp.float32),
        ],
    )(toks, pack, vec)
    return out
